# baseline (speedup 1.0000x reference)
import functools

import jax
import jax.numpy as jnp
from jax import lax
from jax.experimental import pallas as pl
from jax.experimental.pallas import tpu as pltpu
from jax.experimental.pallas import tpu_sc as plsc

N = 10000
E = 320000
D = 128

NPW = 320
CB = 16 * NPW
NPAD = 2 * CB
SHIFT = 16384
MAGIC = 13108
MSH = 22
STRIPE = E // 16
C = 800
ROUNDS = STRIPE // C
SCAP = 896
QCAP = 12800
NEG = float("-inf")
RC = 64
ER = 2500


def _tc_body(x_ref, w_ref, b_ref, a_ref, bm_ref):
    xb = x_ref[...]
    w1 = w_ref[0:D, :]
    w2 = w_ref[D : 2 * D, :]
    a_ref[...] = (
        jnp.dot(xb, w1 - w2, preferred_element_type=jnp.float32) + b_ref[...]
    )
    bm_ref[...] = jnp.dot(xb, w2, preferred_element_type=jnp.float32)


def _node_transforms(xp, W, b2):
    grid = NPAD // 1024
    return pl.pallas_call(
        _tc_body,
        grid=(grid,),
        in_specs=[
            pl.BlockSpec((1024, D), lambda i: (i, 0)),
            pl.BlockSpec((2 * D, D), lambda i: (0, 0)),
            pl.BlockSpec((1, D), lambda i: (0, 0)),
        ],
        out_specs=[
            pl.BlockSpec((1024, D), lambda i: (i, 0)),
            pl.BlockSpec((1024, D), lambda i: (i, 0)),
        ],
        out_shape=[
            jax.ShapeDtypeStruct((NPAD, D), jnp.float32),
            jax.ShapeDtypeStruct((NPAD, D), jnp.float32),
        ],
    )(xp, W, b2)


def _pack_body(s_ref, d_ref, e_ref):
    e_ref[...] = d_ref[...] * SHIFT + s_ref[...]


def _pack_edges(s2, d2):
    return pl.pallas_call(
        _pack_body,
        grid=(1,),
        in_specs=[
            pl.BlockSpec((ER, D), lambda i: (0, 0)),
            pl.BlockSpec((ER, D), lambda i: (0, 0)),
        ],
        out_specs=pl.BlockSpec((ER, D), lambda i: (0, 0)),
        out_shape=jax.ShapeDtypeStruct((ER, D), jnp.int32),
    )(s2, d2)


def _sc_kernel(enc_hbm, bm_hbm, a_hbm, out_hbm, sv, binsf, offs, qstage,
               gidxa, gidxb, rows, accf, astg, ostg, queue, cnts, gsem, gsem2):
    c = lax.axis_index("c")
    s = lax.axis_index("s")
    cb = c * CB
    lo = cb + s * NPW
    elo = lo * SHIFT
    ehi = (lo + NPW) * SHIFT
    ecb = cb * SHIFT
    ecb_hi = (cb + CB) * SHIFT
    qbase = s * QCAP

    iota16 = lax.iota(jnp.int32, 16)

    def init_acc(i, z):
        accf[pl.ds(i * 16, 16)] = jnp.full((16,), NEG, jnp.float32)
        return z

    lax.fori_loop(0, (NPW + 1) * D // 16, init_acc, 0)

    def init_bins(i, z):
        binsf[pl.ds(i * 16, 16)] = jnp.full((16,), -1, jnp.int32)
        return z

    lax.fori_loop(0, 16 * SCAP // 16, init_bins, 0)
    cnts[0] = 0
    plsc.subcore_barrier()

    estr = s * STRIPE

    def round_body(rd, z):
        pltpu.sync_copy(enc_hbm.at[pl.ds(pl.multiple_of(estr + rd * C, 16), C)], sv)
        offs[pl.ds(0, 16)] = jnp.zeros((16,), jnp.int32)
        offs[pl.ds(16, 16)] = jnp.zeros((16,), jnp.int32)

        def svec(i, z2):
            e = sv[pl.ds(i * 16, 16)]
            hit = (e >= ecb) & (e < ecb_hi)
            dl = lax.shift_right_logical(e, 14) - cb
            o = lax.shift_right_logical(dl * MAGIC, MSH)
            o = jnp.where(hit, o, 16)
            rank, last = plsc.scan_count(o, mask=hit)
            base_off = plsc.load_gather(offs, [o])
            pos = base_off + rank
            idx = jnp.where(hit, o * SCAP + pos, 0)
            plsc.store_scatter(binsf, [idx], e, mask=hit)
            plsc.addupdate_scatter(offs, [o], rank + 1, mask=hit & last)
            return z2

        lax.fori_loop(0, C // 16, svec, 0)

        ovec = offs[pl.ds(0, 16)]
        for o in range(16):
            cnt_o = ovec[o]
            r_o = (cnt_o + 15) & -16
            base = plsc.fetch_and_add(cnts.at[0], r_o, subcore_id=o)

            def pub(k, z2, o=o, base=base):
                pltpu.sync_copy(
                    binsf.at[pl.ds(o * SCAP + k * 16, 16)],
                    queue.at[pl.ds(pl.multiple_of(o * QCAP + base + k * 16, 16), 16)],
                )
                return z2

            lax.fori_loop(0, r_o // 16, pub, 0)

        plsc.subcore_barrier()

        T = cnts[0]

        def chunk(k, z2):
            pltpu.sync_copy(queue.at[pl.ds(pl.multiple_of(qbase + k * 128, 128), 128)], qstage)
            for t in range(4):
                gidxa[pl.ds(16 * t, 16)] = jnp.minimum(
                    qstage[pl.ds(16 * t, 16)] & (SHIFT - 1), NPAD - 1
                )
                gidxb[pl.ds(16 * t, 16)] = jnp.minimum(
                    qstage[pl.ds(64 + 16 * t, 16)] & (SHIFT - 1), NPAD - 1
                )
            ca = pltpu.async_copy(bm_hbm.at[gidxa], rows.at[pl.ds(0, 64)], gsem)
            cb2 = pltpu.async_copy(
                bm_hbm.at[gidxb], rows.at[pl.ds(64, 64)], gsem2
            )

            def acc_body(r, z3):
                ev = plsc.load_gather(qstage, [jnp.full((16,), r, jnp.int32)])
                valid = (ev >= elo) & (ev < ehi)
                row = jnp.where(
                    valid, lax.shift_right_logical(ev, 14) - lo, NPW
                )
                base2 = row * D
                for j in range(D // 16):
                    idx = base2 + (16 * j) + iota16
                    cur = plsc.load_gather(accf, [idx])
                    g = rows[r, pl.ds(16 * j, 16)]
                    plsc.store_scatter(accf, [idx], jnp.maximum(cur, g))
                return z3

            nk = jnp.minimum(T - k * 128, 128)
            ca.wait()
            lax.fori_loop(0, jnp.minimum(nk, 64), acc_body, 0)
            cb2.wait()
            lax.fori_loop(64, jnp.maximum(nk, 64), acc_body, 0)
            return z2

        lax.fori_loop(0, (T + 127) // 128, chunk, 0)
        cnts[0] = 0
        plsc.subcore_barrier()
        return z

    lax.fori_loop(0, ROUNDS, round_body, 0)

    for cc in range(NPW // RC):
        pltpu.sync_copy(a_hbm.at[pl.ds(lo + cc * RC, RC)], astg)

        def ep_body(r, z2):
            for j in range(D // 16):
                v = astg[r, pl.ds(16 * j, 16)] + accf[
                    pl.ds((cc * RC + r) * D + 16 * j, 16)
                ]
                ostg[r, pl.ds(16 * j, 16)] = jnp.maximum(v, 0.0)
            return z2

        lax.fori_loop(0, RC, ep_body, 0)
        pltpu.sync_copy(ostg, out_hbm.at[pl.ds(lo + cc * RC, RC)])


_sc_call = functools.partial(
    pl.kernel,
    mesh=plsc.VectorSubcoreMesh(core_axis_name="c", subcore_axis_name="s"),
    out_type=jax.ShapeDtypeStruct((NPAD, D), jnp.float32),
    scratch_types=[
        pltpu.VMEM((C,), jnp.int32),
        pltpu.VMEM((16 * SCAP,), jnp.int32),
        pltpu.VMEM((32,), jnp.int32),
        pltpu.VMEM((128,), jnp.int32),
        pltpu.VMEM((64,), jnp.int32),
        pltpu.VMEM((64,), jnp.int32),
        pltpu.VMEM((128, D), jnp.float32),
        pltpu.VMEM(((NPW + 1) * D,), jnp.float32),
        pltpu.VMEM((RC, D), jnp.float32),
        pltpu.VMEM((RC, D), jnp.float32),
        pltpu.VMEM_SHARED((16 * QCAP,), jnp.int32),
        pltpu.SMEM((8,), jnp.int32),
        pltpu.SemaphoreType.DMA,
        pltpu.SemaphoreType.DMA,
    ],
    compiler_params=pltpu.CompilerParams(needs_layout_passes=False),
)(_sc_kernel)


@jax.jit
def kernel(x, edge_index, W, b):
    xp = jnp.zeros((NPAD, D), jnp.float32).at[:N].set(x)
    s2 = edge_index[0].reshape(ER, D)
    d2 = edge_index[1].reshape(ER, D)
    A, Bm = _node_transforms(xp, W, b.reshape(1, D))
    enc2 = _pack_edges(s2, d2)
    outp = _sc_call(enc2.reshape(E), Bm, A)
    return outp[:N]


# speedup vs baseline: 4.2114x; 4.2114x over previous
import functools

import jax
import jax.numpy as jnp
from jax import lax
from jax.experimental import pallas as pl
from jax.experimental.pallas import tpu as pltpu
from jax.experimental.pallas import tpu_sc as plsc

N = 10000
E = 320000
D = 128

NSUB = 32
NPW = 320
NPAD = NSUB * NPW
SHIFT = 16384
BLK = 6400
NBLK = E // BLK
CAP = 192
G = 128
RC = 64
NEG = float("-inf")
ER = 2500


def _tc_body(x_ref, w_ref, b_ref, a_ref, bm_ref):
    xb = x_ref[...]
    w1 = w_ref[0:D, :]
    w2 = w_ref[D : 2 * D, :]
    a_ref[...] = (
        jnp.dot(xb, w1 - w2, preferred_element_type=jnp.float32) + b_ref[...]
    )
    bm_ref[...] = jnp.dot(xb, w2, preferred_element_type=jnp.float32)


def _node_transforms(xp, W, b2):
    grid = NPAD // 1024
    return pl.pallas_call(
        _tc_body,
        grid=(grid,),
        in_specs=[
            pl.BlockSpec((1024, D), lambda i: (i, 0)),
            pl.BlockSpec((2 * D, D), lambda i: (0, 0)),
            pl.BlockSpec((1, D), lambda i: (0, 0)),
        ],
        out_specs=[
            pl.BlockSpec((1024, D), lambda i: (i, 0)),
            pl.BlockSpec((1024, D), lambda i: (i, 0)),
        ],
        out_shape=[
            jax.ShapeDtypeStruct((NPAD, D), jnp.float32),
            jax.ShapeDtypeStruct((NPAD, D), jnp.float32),
        ],
    )(xp, W, b2)


def _pack_body(s_ref, d_ref, e_ref):
    e_ref[...] = d_ref[...] * SHIFT + s_ref[...]


def _pack_edges(s2, d2):
    return pl.pallas_call(
        _pack_body,
        grid=(1,),
        in_specs=[
            pl.BlockSpec((ER, D), lambda i: (0, 0)),
            pl.BlockSpec((ER, D), lambda i: (0, 0)),
        ],
        out_specs=pl.BlockSpec((ER, D), lambda i: (0, 0)),
        out_shape=jax.ShapeDtypeStruct((ER, D), jnp.int32),
    )(s2, d2)


def _sc_kernel(enc_hbm, bm_hbm, a_hbm, out_hbm, encv, encv2, ebuf, gidxa,
               gidxb, rows, accf, astg, ostg, gsem, gsem2, esem, esem2):
    wid = lax.axis_index("s") * 2 + lax.axis_index("c")
    lo = wid * NPW
    elo = lo * SHIFT
    ehi = (lo + NPW) * SHIFT

    def init_acc(i, c):
        accf[pl.ds(i * 16, 16)] = jnp.full((16,), NEG, jnp.float32)
        return c

    lax.fori_loop(0, NPW * D // 16, init_acc, 0)
    for i in range(CAP // 16):
        ebuf[pl.ds(16 * i, 16)] = jnp.zeros((16,), jnp.int32)

    iota16 = lax.iota(jnp.int32, 16)

    def do_flush(n):
        for t in range(G // 32):
            gidxa[pl.ds(16 * t, 16)] = ebuf[pl.ds(16 * t, 16)] & (SHIFT - 1)
            gidxb[pl.ds(16 * t, 16)] = ebuf[pl.ds(64 + 16 * t, 16)] & (SHIFT - 1)
        ca = pltpu.async_copy(bm_hbm.at[gidxa], rows.at[pl.ds(0, 64)], gsem)
        cb = pltpu.async_copy(bm_hbm.at[gidxb], rows.at[pl.ds(64, 64)], gsem2)

        def acc_body(r, c):
            ev = plsc.load_gather(ebuf, [jnp.full((16,), r, jnp.int32)])
            base = (lax.shift_right_logical(ev, 14) - lo) * D
            for j in range(D // 16):
                idx = base + (16 * j) + iota16
                cur = plsc.load_gather(accf, [idx])
                g = rows[r, pl.ds(16 * j, 16)]
                plsc.store_scatter(accf, [idx], jnp.maximum(cur, g))
            return c

        ca.wait()
        lax.fori_loop(0, jnp.minimum(n, 64), acc_body, 0)
        cb.wait()
        lax.fori_loop(64, jnp.maximum(n, 64), acc_body, 0)

    def scan_buf(buf, m):
        def vec_body(i, m):
            es = [buf[pl.ds(i * 64 + 16 * u, 16)] for u in range(4)]
            msks = [(e >= elo) & (e < ehi) for e in es]
            cnts = [jnp.cumsum(k.astype(jnp.int32)) for k in msks]
            t = m
            for u in range(4):
                plsc.store_scatter(ebuf, [t + cnts[u] - 1], es[u], mask=msks[u])
                t = t + cnts[u][15]

            def fl(mm):
                do_flush(G)
                for u in range(4):
                    ebuf[pl.ds(16 * u, 16)] = ebuf[pl.ds(G + 16 * u, 16)]
                return mm - G

            return lax.cond(t >= G, fl, lambda mm: mm, t)

        return lax.fori_loop(0, BLK // 64, vec_body, m)

    def wait_enc(buf, sem):
        pltpu.make_async_copy(enc_hbm.at[pl.ds(0, BLK)], buf, sem).wait()

    pltpu.async_copy(enc_hbm.at[pl.ds(0, BLK)], encv, esem)

    def blk_body(g, m):
        wait_enc(encv, esem)

        @pl.when(2 * g + 1 < NBLK)
        def _():
            pltpu.async_copy(
                enc_hbm.at[pl.ds((2 * g + 1) * BLK, BLK)], encv2, esem2
            )

        m = scan_buf(encv, m)
        wait_enc(encv2, esem2)

        @pl.when(2 * g + 2 < NBLK)
        def _():
            pltpu.async_copy(
                enc_hbm.at[pl.ds((2 * g + 2) * BLK, BLK)], encv, esem
            )

        return scan_buf(encv2, m)

    m_fin = lax.fori_loop(0, NBLK // 2, blk_body, 0)
    do_flush(m_fin)

    for c in range(NPW // RC):
        pltpu.sync_copy(a_hbm.at[pl.ds(lo + c * RC, RC)], astg)

        def ep_body(r, cc):
            for j in range(D // 16):
                v = astg[r, pl.ds(16 * j, 16)] + accf[
                    pl.ds((c * RC + r) * D + 16 * j, 16)
                ]
                ostg[r, pl.ds(16 * j, 16)] = jnp.maximum(v, 0.0)
            return cc

        lax.fori_loop(0, RC, ep_body, 0)
        pltpu.sync_copy(ostg, out_hbm.at[pl.ds(lo + c * RC, RC)])


_sc_call = functools.partial(
    pl.kernel,
    mesh=plsc.VectorSubcoreMesh(core_axis_name="c", subcore_axis_name="s"),
    out_type=jax.ShapeDtypeStruct((NPAD, D), jnp.float32),
    scratch_types=[
        pltpu.VMEM((BLK,), jnp.int32),
        pltpu.VMEM((BLK,), jnp.int32),
        pltpu.VMEM((CAP,), jnp.int32),
        pltpu.VMEM((G // 2,), jnp.int32),
        pltpu.VMEM((G // 2,), jnp.int32),
        pltpu.VMEM((G, D), jnp.float32),
        pltpu.VMEM((NPW * D,), jnp.float32),
        pltpu.VMEM((RC, D), jnp.float32),
        pltpu.VMEM((RC, D), jnp.float32),
        pltpu.SemaphoreType.DMA,
        pltpu.SemaphoreType.DMA,
        pltpu.SemaphoreType.DMA,
        pltpu.SemaphoreType.DMA,
    ],
    compiler_params=pltpu.CompilerParams(needs_layout_passes=False),
)(_sc_kernel)


@jax.jit
def kernel(x, edge_index, W, b):
    xp = jnp.zeros((NPAD, D), jnp.float32).at[:N].set(x)
    s2 = edge_index[0].reshape(ER, D)
    d2 = edge_index[1].reshape(ER, D)
    A, Bm = _node_transforms(xp, W, b.reshape(1, D))
    enc2 = _pack_edges(s2, d2)
    outp = _sc_call(enc2.reshape(E), Bm, A)
    return outp[:N]


# speedup vs baseline: 4.2855x; 1.0176x over previous
import functools

import jax
import jax.numpy as jnp
from jax import lax
from jax.experimental import pallas as pl
from jax.experimental.pallas import tpu as pltpu
from jax.experimental.pallas import tpu_sc as plsc

N = 10000
E = 320000
D = 128

NSUB = 32
NPW = 320
NPAD = NSUB * NPW
SHIFT = 16384
BLK = 6400
NBLK = E // BLK
CAP = 256
G = 128
RC = 64
NEG = float("-inf")
ER = 2500


def _tc_body(x_ref, w_ref, b_ref, a_ref, bm_ref):
    xb = x_ref[...]
    w1 = w_ref[0:D, :]
    w2 = w_ref[D : 2 * D, :]
    a_ref[...] = (
        jnp.dot(xb, w1 - w2, preferred_element_type=jnp.float32) + b_ref[...]
    )
    bm_ref[...] = jnp.dot(xb, w2, preferred_element_type=jnp.float32)


def _node_transforms(xp, W, b2):
    grid = NPAD // 1024
    return pl.pallas_call(
        _tc_body,
        grid=(grid,),
        in_specs=[
            pl.BlockSpec((1024, D), lambda i: (i, 0)),
            pl.BlockSpec((2 * D, D), lambda i: (0, 0)),
            pl.BlockSpec((1, D), lambda i: (0, 0)),
        ],
        out_specs=[
            pl.BlockSpec((1024, D), lambda i: (i, 0)),
            pl.BlockSpec((1024, D), lambda i: (i, 0)),
        ],
        out_shape=[
            jax.ShapeDtypeStruct((NPAD, D), jnp.float32),
            jax.ShapeDtypeStruct((NPAD, D), jnp.float32),
        ],
    )(xp, W, b2)


def _pack_body(s_ref, d_ref, e_ref):
    e_ref[...] = d_ref[...] * SHIFT + s_ref[...]


def _pack_edges(s2, d2):
    return pl.pallas_call(
        _pack_body,
        grid=(1,),
        in_specs=[
            pl.BlockSpec((ER, D), lambda i: (0, 0)),
            pl.BlockSpec((ER, D), lambda i: (0, 0)),
        ],
        out_specs=pl.BlockSpec((ER, D), lambda i: (0, 0)),
        out_shape=jax.ShapeDtypeStruct((ER, D), jnp.int32),
    )(s2, d2)


def _sc_kernel(enc_hbm, bm_hbm, a_hbm, out_hbm, encv, encv2, ebuf, gidxa,
               gidxb, rows, accf, astg, ostg, gsem, gsem2, esem, esem2):
    wid = lax.axis_index("s") * 2 + lax.axis_index("c")
    lo = wid * NPW
    elo = lo * SHIFT
    ehi = (lo + NPW) * SHIFT

    def init_acc(i, c):
        accf[pl.ds(i * 16, 16)] = jnp.full((16,), NEG, jnp.float32)
        return c

    lax.fori_loop(0, NPW * D // 16, init_acc, 0)
    for i in range(CAP // 16):
        ebuf[pl.ds(16 * i, 16)] = jnp.zeros((16,), jnp.int32)

    iota16 = lax.iota(jnp.int32, 16)

    def do_flush(n):
        for t in range(G // 32):
            gidxa[pl.ds(16 * t, 16)] = ebuf[pl.ds(16 * t, 16)] & (SHIFT - 1)
            gidxb[pl.ds(16 * t, 16)] = ebuf[pl.ds(64 + 16 * t, 16)] & (SHIFT - 1)
        ca = pltpu.async_copy(bm_hbm.at[gidxa], rows.at[pl.ds(0, 64)], gsem)
        cb = pltpu.async_copy(bm_hbm.at[gidxb], rows.at[pl.ds(64, 64)], gsem2)

        def acc_body(r, c):
            ev = plsc.load_gather(ebuf, [jnp.full((16,), r, jnp.int32)])
            base = (lax.shift_right_logical(ev, 14) - lo) * D
            for j in range(D // 16):
                idx = base + (16 * j) + iota16
                cur = plsc.load_gather(accf, [idx])
                g = rows[r, pl.ds(16 * j, 16)]
                plsc.store_scatter(accf, [idx], jnp.maximum(cur, g))
            return c

        ca.wait()
        lax.fori_loop(0, jnp.minimum(n, 64), acc_body, 0)
        cb.wait()
        lax.fori_loop(64, jnp.maximum(n, 64), acc_body, 0)

    def scan_buf(buf, m):
        def vec_body(i, m):
            es = [buf[pl.ds(i * 128 + 16 * u, 16)] for u in range(8)]
            msks = [(e >= elo) & (e < ehi) for e in es]
            cnts = [jnp.cumsum(k.astype(jnp.int32)) for k in msks]
            t = m
            for u in range(8):
                plsc.store_scatter(ebuf, [t + cnts[u] - 1], es[u], mask=msks[u])
                t = t + cnts[u][15]

            def fl(mm):
                do_flush(G)
                for u in range(8):
                    ebuf[pl.ds(16 * u, 16)] = ebuf[pl.ds(G + 16 * u, 16)]
                return mm - G

            return lax.cond(t >= G, fl, lambda mm: mm, t)

        return lax.fori_loop(0, BLK // 128, vec_body, m)

    def wait_enc(buf, sem):
        pltpu.make_async_copy(enc_hbm.at[pl.ds(0, BLK)], buf, sem).wait()

    pltpu.async_copy(enc_hbm.at[pl.ds(0, BLK)], encv, esem)

    def blk_body(g, m):
        wait_enc(encv, esem)

        @pl.when(2 * g + 1 < NBLK)
        def _():
            pltpu.async_copy(
                enc_hbm.at[pl.ds((2 * g + 1) * BLK, BLK)], encv2, esem2
            )

        m = scan_buf(encv, m)
        wait_enc(encv2, esem2)

        @pl.when(2 * g + 2 < NBLK)
        def _():
            pltpu.async_copy(
                enc_hbm.at[pl.ds((2 * g + 2) * BLK, BLK)], encv, esem
            )

        return scan_buf(encv2, m)

    m_fin = lax.fori_loop(0, NBLK // 2, blk_body, 0)
    do_flush(m_fin)

    for c in range(NPW // RC):
        pltpu.sync_copy(a_hbm.at[pl.ds(lo + c * RC, RC)], astg)

        def ep_body(r, cc):
            for j in range(D // 16):
                v = astg[r, pl.ds(16 * j, 16)] + accf[
                    pl.ds((c * RC + r) * D + 16 * j, 16)
                ]
                ostg[r, pl.ds(16 * j, 16)] = jnp.maximum(v, 0.0)
            return cc

        lax.fori_loop(0, RC, ep_body, 0)
        pltpu.sync_copy(ostg, out_hbm.at[pl.ds(lo + c * RC, RC)])


_sc_call = functools.partial(
    pl.kernel,
    mesh=plsc.VectorSubcoreMesh(core_axis_name="c", subcore_axis_name="s"),
    out_type=jax.ShapeDtypeStruct((NPAD, D), jnp.float32),
    scratch_types=[
        pltpu.VMEM((BLK,), jnp.int32),
        pltpu.VMEM((BLK,), jnp.int32),
        pltpu.VMEM((CAP,), jnp.int32),
        pltpu.VMEM((G // 2,), jnp.int32),
        pltpu.VMEM((G // 2,), jnp.int32),
        pltpu.VMEM((G, D), jnp.float32),
        pltpu.VMEM((NPW * D,), jnp.float32),
        pltpu.VMEM((RC, D), jnp.float32),
        pltpu.VMEM((RC, D), jnp.float32),
        pltpu.SemaphoreType.DMA,
        pltpu.SemaphoreType.DMA,
        pltpu.SemaphoreType.DMA,
        pltpu.SemaphoreType.DMA,
    ],
    compiler_params=pltpu.CompilerParams(needs_layout_passes=False),
)(_sc_kernel)


@jax.jit
def kernel(x, edge_index, W, b):
    xp = jnp.zeros((NPAD, D), jnp.float32).at[:N].set(x)
    s2 = edge_index[0].reshape(ER, D)
    d2 = edge_index[1].reshape(ER, D)
    A, Bm = _node_transforms(xp, W, b.reshape(1, D))
    enc2 = _pack_edges(s2, d2)
    outp = _sc_call(enc2.reshape(E), Bm, A)
    return outp[:N]
